# baseline (speedup 1.0000x reference)
import functools

import jax
import jax.numpy as jnp
from jax import lax
from jax.experimental import pallas as pl
from jax.experimental.pallas import tpu as pltpu
from jax.experimental.pallas import tpu_sc as plsc

_N = 100000
_E = 3200000
_REST_MULT = 1.0
_D = 16
_SKEW = 17
_NW = 32
_EPW = _E // _NW
_C = 400
_NCHUNK = _EPW // _C
_NSUB = 5
_CSUB = _C // _NSUB


_CN = 2000
_NCH_TP = _N // _CN
_TP_ROUNDS = (_NCH_TP + _NW - 1) // _NW


def _make_pack_kernel():
    mesh = plsc.VectorSubcoreMesh(core_axis_name="c", subcore_axis_name="s")

    @functools.partial(
        pl.kernel,
        mesh=mesh,
        compiler_params=pltpu.CompilerParams(
            use_tc_tiling_on_sc=False, needs_layout_passes=False),
        out_type=(
            jax.ShapeDtypeStruct((_N, _D), jnp.float32),
            jax.ShapeDtypeStruct((_N, _D), jnp.float32),
        ),
        scratch_types=[
            pltpu.VMEM((3 * _CN,), jnp.float32),
            pltpu.VMEM((3 * _CN,), jnp.float32),
            pltpu.VMEM((_CN,), jnp.float32),
            pltpu.VMEM((_CN,), jnp.float32),
            pltpu.VMEM((_CN,), jnp.float32),
            pltpu.VMEM((_SKEW * _CN,), jnp.float32),
            pltpu.VMEM((_CN, _D), jnp.float32),
            pltpu.VMEM((_CN, _D), jnp.float32),
            pltpu.SemaphoreType.DMA,
            pltpu.SemaphoreType.DMA,
        ],
    )
    def table_pack(pos_hbm, rest_hbm, mu_hbm, lam_hbm, bend_hbm,
                   ta_hbm, tb_hbm,
                   pos_v, rest_v, mu_v, lam_v, bend_v,
                   ska, ta_v, tb_v, sem_in, sem_out):
        wid = lax.axis_index("s") * 2 + lax.axis_index("c")
        iota = lax.iota(jnp.int32, 16)
        pat3 = iota * 3
        pat17 = iota * _SKEW
        sgn = jnp.where((iota >= 3) & (iota < 9), -1.0, 1.0)

        def do_chunk(c):
            base = c * _CN
            pltpu.async_copy(pos_hbm.at[pl.ds(3 * base, 3 * _CN)], pos_v, sem_in)
            pltpu.async_copy(rest_hbm.at[pl.ds(3 * base, 3 * _CN)], rest_v, sem_in)
            pltpu.async_copy(mu_hbm.at[pl.ds(base, _CN)], mu_v, sem_in)
            pltpu.async_copy(lam_hbm.at[pl.ds(base, _CN)], lam_v, sem_in)
            pltpu.async_copy(bend_hbm.at[pl.ds(base, _CN)], bend_v, sem_in)
            pltpu.make_async_copy(pos_hbm.at[pl.ds(0, 3 * _CN)], pos_v, sem_in).wait()
            pltpu.make_async_copy(rest_hbm.at[pl.ds(0, 3 * _CN)], rest_v, sem_in).wait()
            pltpu.make_async_copy(mu_hbm.at[pl.ds(0, _CN)], mu_v, sem_in).wait()
            pltpu.make_async_copy(lam_hbm.at[pl.ds(0, _CN)], lam_v, sem_in).wait()
            pltpu.make_async_copy(bend_hbm.at[pl.ds(0, _CN)], bend_v, sem_in).wait()

            @plsc.parallel_loop(0, _CN // 16, step=1, unroll=2)
            def _(q):
                nb = q * 16
                skoff = q * (16 * _SKEW) + pat17
                for f, src in ((0, mu_v), (1, lam_v), (2, bend_v)):
                    v = src[pl.ds(nb, 16)] * 0.5
                    plsc.store_scatter(ska, [skoff + f], v)
                for c3 in range(3):
                    vp = plsc.load_gather(pos_v, [q * 48 + pat3 + c3])
                    plsc.store_scatter(ska, [skoff + 3 + c3], vp)
                    vr = plsc.load_gather(rest_v, [q * 48 + pat3 + c3]) * _REST_MULT
                    plsc.store_scatter(ska, [skoff + 6 + c3], vr)
                zero = vp * 0.0
                for f in range(9, _D):
                    plsc.store_scatter(ska, [skoff + f], zero)

            @plsc.parallel_loop(0, _CN, step=1, unroll=8)
            def _(j):
                v = plsc.load_gather(ska, [j * _SKEW + iota])
                ta_v[j] = v
                tb_v[j] = v * sgn

            pltpu.async_copy(ta_v, ta_hbm.at[pl.ds(base, _CN)], sem_out)
            pltpu.async_copy(tb_v, tb_hbm.at[pl.ds(base, _CN)], sem_out)
            pltpu.make_async_copy(ta_v, ta_hbm.at[pl.ds(0, _CN)], sem_out).wait()
            pltpu.make_async_copy(tb_v, tb_hbm.at[pl.ds(0, _CN)], sem_out).wait()

        def round_body(t, carry):
            c = wid + t * _NW

            @pl.when(c < _NCH_TP)
            def _():
                do_chunk(c)

            return carry

        lax.fori_loop(0, _TP_ROUNDS, round_body, 0)

    return table_pack


def _make_gather_kernel():
    mesh = plsc.VectorSubcoreMesh(core_axis_name="c", subcore_axis_name="s")

    @functools.partial(
        pl.kernel,
        mesh=mesh,
        compiler_params=pltpu.CompilerParams(
            use_tc_tiling_on_sc=False, needs_layout_passes=False),
        out_type=(
            jax.ShapeDtypeStruct((_E,), jnp.float32),
            jax.ShapeDtypeStruct((_E,), jnp.float32),
            jax.ShapeDtypeStruct((_E,), jnp.float32),
            jax.ShapeDtypeStruct((4 * _E,), jnp.float32),
            jax.ShapeDtypeStruct((4 * _E,), jnp.float32),
        ),
        scratch_types=[
            pltpu.VMEM((_C,), jnp.int32), pltpu.VMEM((_C,), jnp.int32),
            pltpu.VMEM((_C,), jnp.int32), pltpu.VMEM((_C,), jnp.int32),
            pltpu.VMEM((_C, _D), jnp.float32), pltpu.VMEM((_C, _D), jnp.float32),
            pltpu.VMEM((_C, _D), jnp.float32), pltpu.VMEM((_C, _D), jnp.float32),
            pltpu.VMEM((_SKEW * _C,), jnp.float32),
            pltpu.VMEM((_C,), jnp.float32), pltpu.VMEM((_C,), jnp.float32),
            pltpu.VMEM((_C,), jnp.float32), pltpu.VMEM((_C,), jnp.float32),
            pltpu.VMEM((_C,), jnp.float32), pltpu.VMEM((_C,), jnp.float32),
            pltpu.VMEM((4 * _C,), jnp.float32), pltpu.VMEM((4 * _C,), jnp.float32),
            pltpu.VMEM((4 * _C,), jnp.float32), pltpu.VMEM((4 * _C,), jnp.float32),
        ] + [pltpu.SemaphoreType.DMA] * 10,
    )
    def edge_gather(ta_hbm, tb_hbm, ei_hbm,
                    mu_hbm, lam_hbm, bend_hbm, rr_hbm, rp_hbm,
                    is0, is1, id0, id1, ra0, ra1, rb0, rb1, skb,
                    mu0, mu1, la0, la1, be0, be1, rp0, rp1, rr0, rr1,
                    sis0, sis1, sid0, sid1, sga0, sga1, sgb0, sgb1,
                    swb0, swb1):
        idx_s = (is0, is1)
        idx_d = (id0, id1)
        ra = (ra0, ra1)
        rb = (rb0, rb1)
        mu_st = (mu0, mu1)
        lam_st = (la0, la1)
        bend_st = (be0, be1)
        rp_st = (rp0, rp1)
        rr_st = (rr0, rr1)
        sis = (sis0, sis1)
        sid = (sid0, sid1)
        sga = (sga0, sga1)
        sgb = (sgb0, sgb1)
        swb = (swb0, swb1)

        wid = lax.axis_index("s") * 2 + lax.axis_index("c")
        base_w = wid * _EPW

        iota = lax.iota(jnp.int32, 16)
        skew_pat = iota * _SKEW
        pat4 = iota * 4

        def issue_idx(g, p):
            base = base_w + g * _C
            pltpu.async_copy(ei_hbm.at[pl.ds(base, _C)], idx_s[p], sis[p])
            pltpu.async_copy(ei_hbm.at[pl.ds(_E + base, _C)], idx_d[p], sid[p])

        def wait_idx(p):
            pltpu.make_async_copy(ei_hbm.at[pl.ds(0, _C)], idx_s[p], sis[p]).wait()
            pltpu.make_async_copy(ei_hbm.at[pl.ds(0, _C)], idx_d[p], sid[p]).wait()

        def issue_gathers(p):
            for s in range(_NSUB):
                sl = pl.ds(s * _CSUB, _CSUB)
                pltpu.async_copy(ta_hbm.at[idx_s[p].at[sl]], ra[p].at[sl], sga[p])
                pltpu.async_copy(tb_hbm.at[idx_d[p].at[sl]], rb[p].at[sl], sgb[p])

        def wait_gathers(p):
            for s in range(_NSUB):
                sl = pl.ds(s * _CSUB, _CSUB)
                pltpu.make_async_copy(ta_hbm.at[idx_s[p].at[sl]], ra[p].at[sl], sga[p]).wait()
                pltpu.make_async_copy(tb_hbm.at[idx_d[p].at[sl]], rb[p].at[sl], sgb[p]).wait()

        def wb_descs(g, p):
            base = base_w + g * _C
            return (
                pltpu.make_async_copy(mu_st[p], mu_hbm.at[pl.ds(base, _C)], swb[p]),
                pltpu.make_async_copy(lam_st[p], lam_hbm.at[pl.ds(base, _C)], swb[p]),
                pltpu.make_async_copy(bend_st[p], bend_hbm.at[pl.ds(base, _C)], swb[p]),
                pltpu.make_async_copy(rp_st[p], rp_hbm.at[pl.ds(4 * base, 4 * _C)], swb[p]),
                pltpu.make_async_copy(rr_st[p], rr_hbm.at[pl.ds(4 * base, 4 * _C)], swb[p]),
            )

        def issue_wb(g, p):
            for d in wb_descs(g, p):
                d.start()

        def wait_wb(p):
            for d in wb_descs(0, p):
                d.wait()

        issue_idx(0, 0)
        issue_idx(1, 1)
        wait_idx(0)
        issue_gathers(0)

        def outer_body(go, carry):
            for b in (0, 1):
                g = 2 * go + b
                wait_gathers(b)

                @pl.when(g + 2 < _NCHUNK)
                def _():
                    issue_idx(g + 2, b)

                @pl.when(g >= 1)
                def _():
                    wait_wb(1 - b)

                @pl.when(g + 1 < _NCHUNK)
                def _():
                    wait_idx(1 - b)
                    issue_gathers(1 - b)

                rap = ra[b]
                rbp = rb[b]

                @plsc.parallel_loop(0, _C, step=1, unroll=8)
                def _(j):
                    v = rap[j] + rbp[j]
                    plsc.store_scatter(skb, [j * _SKEW + iota], v)

                mup = mu_st[b]
                lamp = lam_st[b]
                bendp = bend_st[b]
                rpp = rp_st[b]
                rrp = rr_st[b]

                @plsc.parallel_loop(0, _C // 16, step=1, unroll=2)
                def _(q):
                    gbase = q * (16 * _SKEW) + skew_pat
                    obase = q * 16
                    mup[pl.ds(obase, 16)] = plsc.load_gather(skb, [gbase])
                    lamp[pl.ds(obase, 16)] = plsc.load_gather(skb, [gbase + 1])
                    bendp[pl.ds(obase, 16)] = plsc.load_gather(skb, [gbase + 2])
                    tri = q * 64 + pat4
                    for c in range(3):
                        vpos = plsc.load_gather(skb, [gbase + (3 + c)])
                        plsc.store_scatter(rpp, [tri + c], vpos)
                        vrest = plsc.load_gather(skb, [gbase + (6 + c)])
                        plsc.store_scatter(rrp, [tri + c], vrest)

                issue_wb(g, b)
            return carry

        lax.fori_loop(0, _NCHUNK // 2, outer_body, 0)
        wait_wb(1)

    return edge_gather


_gather_call_cache = []


def kernel(pos, rest_pos, lame_mu_input, lame_lambda_input,
           bending_coeff_input, edge_index):
    if not _gather_call_cache:
        _gather_call_cache.append((_make_pack_kernel(), _make_gather_kernel()))
    _pack_call, _gather_call = _gather_call_cache[0]
    ta, tb = _pack_call(
        pos.reshape(3 * _N),
        rest_pos.reshape(3 * _N),
        lame_mu_input.reshape(_N),
        lame_lambda_input.reshape(_N),
        bending_coeff_input.reshape(_N),
    )
    mu_e, lam_e, bend_e, rr3, rp3 = _gather_call(
        ta, tb, edge_index.reshape(2 * _E))
    return (
        mu_e.reshape(_E, 1),
        lam_e.reshape(_E, 1),
        bend_e.reshape(_E, 1),
        rr3.reshape(_E, 4)[:, :3],
        rp3.reshape(_E, 4)[:, :3],
    )


# speedup vs baseline: 1.4238x; 1.4238x over previous
import functools

import jax
import jax.numpy as jnp
from jax import lax
from jax.experimental import pallas as pl
from jax.experimental.pallas import tpu as pltpu
from jax.experimental.pallas import tpu_sc as plsc

_N = 100000
_E = 3200000
_REST_MULT = 1.0
_D = 16
_SKEW = 17
_NW = 32
_EPW = _E // _NW
_C = 400
_NCHUNK = _EPW // _C
_NSUB = 5
_CSUB = _C // _NSUB


_CN = 2000
_NCH_TP = _N // _CN
_TP_ROUNDS = (_NCH_TP + _NW - 1) // _NW


def _make_pack_kernel():
    mesh = plsc.VectorSubcoreMesh(core_axis_name="c", subcore_axis_name="s")

    @functools.partial(
        pl.kernel,
        mesh=mesh,
        compiler_params=pltpu.CompilerParams(
            use_tc_tiling_on_sc=False, needs_layout_passes=False),
        out_type=(
            jax.ShapeDtypeStruct((_N, _D), jnp.float32),
            jax.ShapeDtypeStruct((_N, _D), jnp.float32),
        ),
        scratch_types=[
            pltpu.VMEM((3 * _CN,), jnp.float32),
            pltpu.VMEM((3 * _CN,), jnp.float32),
            pltpu.VMEM((_CN,), jnp.float32),
            pltpu.VMEM((_CN,), jnp.float32),
            pltpu.VMEM((_CN,), jnp.float32),
            pltpu.VMEM((_SKEW * _CN,), jnp.float32),
            pltpu.VMEM((_CN, _D), jnp.float32),
            pltpu.VMEM((_CN, _D), jnp.float32),
            pltpu.SemaphoreType.DMA,
            pltpu.SemaphoreType.DMA,
        ],
    )
    def table_pack(pos_hbm, rest_hbm, mu_hbm, lam_hbm, bend_hbm,
                   ta_hbm, tb_hbm,
                   pos_v, rest_v, mu_v, lam_v, bend_v,
                   ska, ta_v, tb_v, sem_in, sem_out):
        wid = lax.axis_index("s") * 2 + lax.axis_index("c")
        iota = lax.iota(jnp.int32, 16)
        pat3 = iota * 3
        pat17 = iota * _SKEW
        sgn = jnp.where((iota >= 3) & (iota < 9), -1.0, 1.0)

        def do_chunk(c):
            base = c * _CN
            pltpu.async_copy(pos_hbm.at[pl.ds(3 * base, 3 * _CN)], pos_v, sem_in)
            pltpu.async_copy(rest_hbm.at[pl.ds(3 * base, 3 * _CN)], rest_v, sem_in)
            pltpu.async_copy(mu_hbm.at[pl.ds(base, _CN)], mu_v, sem_in)
            pltpu.async_copy(lam_hbm.at[pl.ds(base, _CN)], lam_v, sem_in)
            pltpu.async_copy(bend_hbm.at[pl.ds(base, _CN)], bend_v, sem_in)
            pltpu.make_async_copy(pos_hbm.at[pl.ds(0, 3 * _CN)], pos_v, sem_in).wait()
            pltpu.make_async_copy(rest_hbm.at[pl.ds(0, 3 * _CN)], rest_v, sem_in).wait()
            pltpu.make_async_copy(mu_hbm.at[pl.ds(0, _CN)], mu_v, sem_in).wait()
            pltpu.make_async_copy(lam_hbm.at[pl.ds(0, _CN)], lam_v, sem_in).wait()
            pltpu.make_async_copy(bend_hbm.at[pl.ds(0, _CN)], bend_v, sem_in).wait()

            @plsc.parallel_loop(0, _CN // 16, step=1, unroll=2)
            def _(q):
                nb = q * 16
                skoff = q * (16 * _SKEW) + pat17
                for f, src in ((0, mu_v), (1, lam_v), (2, bend_v)):
                    v = src[pl.ds(nb, 16)] * 0.5
                    plsc.store_scatter(ska, [skoff + f], v)
                for c3 in range(3):
                    vp = plsc.load_gather(pos_v, [q * 48 + pat3 + c3])
                    plsc.store_scatter(ska, [skoff + 3 + c3], vp)
                    vr = plsc.load_gather(rest_v, [q * 48 + pat3 + c3]) * _REST_MULT
                    plsc.store_scatter(ska, [skoff + 6 + c3], vr)
                zero = vp * 0.0
                for f in range(9, _D):
                    plsc.store_scatter(ska, [skoff + f], zero)

            @plsc.parallel_loop(0, _CN, step=1, unroll=8)
            def _(j):
                v = plsc.load_gather(ska, [j * _SKEW + iota])
                ta_v[j] = v
                tb_v[j] = v * sgn

            pltpu.async_copy(ta_v, ta_hbm.at[pl.ds(base, _CN)], sem_out)
            pltpu.async_copy(tb_v, tb_hbm.at[pl.ds(base, _CN)], sem_out)
            pltpu.make_async_copy(ta_v, ta_hbm.at[pl.ds(0, _CN)], sem_out).wait()
            pltpu.make_async_copy(tb_v, tb_hbm.at[pl.ds(0, _CN)], sem_out).wait()

        def round_body(t, carry):
            c = wid + t * _NW

            @pl.when(c < _NCH_TP)
            def _():
                do_chunk(c)

            return carry

        lax.fori_loop(0, _TP_ROUNDS, round_body, 0)

    return table_pack


def _make_gather_kernel():
    mesh = plsc.VectorSubcoreMesh(core_axis_name="c", subcore_axis_name="s")

    @functools.partial(
        pl.kernel,
        mesh=mesh,
        compiler_params=pltpu.CompilerParams(
            use_tc_tiling_on_sc=False, needs_layout_passes=False),
        out_type=(
            jax.ShapeDtypeStruct((_E,), jnp.float32),
            jax.ShapeDtypeStruct((_E,), jnp.float32),
            jax.ShapeDtypeStruct((_E,), jnp.float32),
            jax.ShapeDtypeStruct((_E, 3), jnp.float32),
            jax.ShapeDtypeStruct((_E, 3), jnp.float32),
        ),
        scratch_types=[
            pltpu.VMEM((_C,), jnp.int32), pltpu.VMEM((_C,), jnp.int32),
            pltpu.VMEM((_C,), jnp.int32), pltpu.VMEM((_C,), jnp.int32),
            pltpu.VMEM((_C, _D), jnp.float32), pltpu.VMEM((_C, _D), jnp.float32),
            pltpu.VMEM((_C, _D), jnp.float32), pltpu.VMEM((_C, _D), jnp.float32),
            pltpu.VMEM((_SKEW * _C,), jnp.float32),
            pltpu.VMEM((_C,), jnp.float32), pltpu.VMEM((_C,), jnp.float32),
            pltpu.VMEM((_C,), jnp.float32), pltpu.VMEM((_C,), jnp.float32),
            pltpu.VMEM((_C,), jnp.float32), pltpu.VMEM((_C,), jnp.float32),
            pltpu.VMEM((_C, 3), jnp.float32), pltpu.VMEM((_C, 3), jnp.float32),
            pltpu.VMEM((_C, 3), jnp.float32), pltpu.VMEM((_C, 3), jnp.float32),
        ] + [pltpu.SemaphoreType.DMA] * 10,
    )
    def edge_gather(ta_hbm, tb_hbm, ei_hbm,
                    mu_hbm, lam_hbm, bend_hbm, rr_hbm, rp_hbm,
                    is0, is1, id0, id1, ra0, ra1, rb0, rb1, skb,
                    mu0, mu1, la0, la1, be0, be1, rp0, rp1, rr0, rr1,
                    sis0, sis1, sid0, sid1, sga0, sga1, sgb0, sgb1,
                    swb0, swb1):
        idx_s = (is0, is1)
        idx_d = (id0, id1)
        ra = (ra0, ra1)
        rb = (rb0, rb1)
        mu_st = (mu0, mu1)
        lam_st = (la0, la1)
        bend_st = (be0, be1)
        rp_st = (rp0, rp1)
        rr_st = (rr0, rr1)
        sis = (sis0, sis1)
        sid = (sid0, sid1)
        sga = (sga0, sga1)
        sgb = (sgb0, sgb1)
        swb = (swb0, swb1)

        wid = lax.axis_index("s") * 2 + lax.axis_index("c")
        base_w = wid * _EPW

        iota = lax.iota(jnp.int32, 16)
        skew_pat = iota * _SKEW
        pat3 = iota * 3

        def issue_idx(g, p):
            base = base_w + g * _C
            pltpu.async_copy(ei_hbm.at[pl.ds(base, _C)], idx_s[p], sis[p])
            pltpu.async_copy(ei_hbm.at[pl.ds(_E + base, _C)], idx_d[p], sid[p])

        def wait_idx(p):
            pltpu.make_async_copy(ei_hbm.at[pl.ds(0, _C)], idx_s[p], sis[p]).wait()
            pltpu.make_async_copy(ei_hbm.at[pl.ds(0, _C)], idx_d[p], sid[p]).wait()

        def issue_gathers(p):
            for s in range(_NSUB):
                sl = pl.ds(s * _CSUB, _CSUB)
                pltpu.async_copy(ta_hbm.at[idx_s[p].at[sl]], ra[p].at[sl], sga[p])
                pltpu.async_copy(tb_hbm.at[idx_d[p].at[sl]], rb[p].at[sl], sgb[p])

        def wait_gathers(p):
            for s in range(_NSUB):
                sl = pl.ds(s * _CSUB, _CSUB)
                pltpu.make_async_copy(ta_hbm.at[idx_s[p].at[sl]], ra[p].at[sl], sga[p]).wait()
                pltpu.make_async_copy(tb_hbm.at[idx_d[p].at[sl]], rb[p].at[sl], sgb[p]).wait()

        def wb_descs(g, p):
            base = base_w + g * _C
            return (
                pltpu.make_async_copy(mu_st[p], mu_hbm.at[pl.ds(base, _C)], swb[p]),
                pltpu.make_async_copy(lam_st[p], lam_hbm.at[pl.ds(base, _C)], swb[p]),
                pltpu.make_async_copy(bend_st[p], bend_hbm.at[pl.ds(base, _C)], swb[p]),
                pltpu.make_async_copy(rp_st[p], rp_hbm.at[pl.ds(base, _C)], swb[p]),
                pltpu.make_async_copy(rr_st[p], rr_hbm.at[pl.ds(base, _C)], swb[p]),
            )

        def issue_wb(g, p):
            for d in wb_descs(g, p):
                d.start()

        def wait_wb(p):
            for d in wb_descs(0, p):
                d.wait()

        issue_idx(0, 0)
        issue_idx(1, 1)
        wait_idx(0)
        issue_gathers(0)

        def outer_body(go, carry):
            for b in (0, 1):
                g = 2 * go + b
                wait_gathers(b)

                @pl.when(g + 2 < _NCHUNK)
                def _():
                    issue_idx(g + 2, b)

                @pl.when(g >= 1)
                def _():
                    wait_wb(1 - b)

                @pl.when(g + 1 < _NCHUNK)
                def _():
                    wait_idx(1 - b)
                    issue_gathers(1 - b)

                rap = ra[b]
                rbp = rb[b]

                @plsc.parallel_loop(0, _C, step=1, unroll=8)
                def _(j):
                    v = rap[j] + rbp[j]
                    plsc.store_scatter(skb, [j * _SKEW + iota], v)

                mup = mu_st[b]
                lamp = lam_st[b]
                bendp = bend_st[b]
                rpp = rp_st[b]
                rrp = rr_st[b]

                @plsc.parallel_loop(0, _C // 16, step=1, unroll=2)
                def _(q):
                    gbase = q * (16 * _SKEW) + skew_pat
                    obase = q * 16
                    mup[pl.ds(obase, 16)] = plsc.load_gather(skb, [gbase])
                    lamp[pl.ds(obase, 16)] = plsc.load_gather(skb, [gbase + 1])
                    bendp[pl.ds(obase, 16)] = plsc.load_gather(skb, [gbase + 2])
                    rows16 = q * 16 + iota
                    for c in range(3):
                        colc = iota * 0 + c
                        vpos = plsc.load_gather(skb, [gbase + (3 + c)])
                        plsc.store_scatter(rpp, [rows16, colc], vpos)
                        vrest = plsc.load_gather(skb, [gbase + (6 + c)])
                        plsc.store_scatter(rrp, [rows16, colc], vrest)

                issue_wb(g, b)
            return carry

        lax.fori_loop(0, _NCHUNK // 2, outer_body, 0)
        wait_wb(1)

    return edge_gather


_gather_call_cache = []


def kernel(pos, rest_pos, lame_mu_input, lame_lambda_input,
           bending_coeff_input, edge_index):
    if not _gather_call_cache:
        _gather_call_cache.append((_make_pack_kernel(), _make_gather_kernel()))
    _pack_call, _gather_call = _gather_call_cache[0]
    ta, tb = _pack_call(
        pos.reshape(3 * _N),
        rest_pos.reshape(3 * _N),
        lame_mu_input.reshape(_N),
        lame_lambda_input.reshape(_N),
        bending_coeff_input.reshape(_N),
    )
    mu_e, lam_e, bend_e, rr3, rp3 = _gather_call(
        ta, tb, edge_index.reshape(2 * _E))
    return (
        mu_e.reshape(_E, 1),
        lam_e.reshape(_E, 1),
        bend_e.reshape(_E, 1),
        rr3,
        rp3,
    )


# speedup vs baseline: 8.8124x; 6.1892x over previous
import functools

import jax
import jax.numpy as jnp
from jax import lax
from jax.experimental import pallas as pl
from jax.experimental.pallas import tpu as pltpu
from jax.experimental.pallas import tpu_sc as plsc

_N = 100000
_E = 3200000
_REST_MULT = 1.0
_D = 16
_SKEW = 17
_NW = 32
_EPW = _E // _NW
_C = 512
_BPC = _C // 128
_NCH = _E // _C
_NT = (_NCH + _NW - 1) // _NW
_NSUB = 4
_CSUB = _C // _NSUB


_CN = 2000
_NCH_TP = _N // _CN
_TP_ROUNDS = (_NCH_TP + _NW - 1) // _NW


def _make_pack_kernel():
    mesh = plsc.VectorSubcoreMesh(core_axis_name="c", subcore_axis_name="s")

    @functools.partial(
        pl.kernel,
        mesh=mesh,
        compiler_params=pltpu.CompilerParams(
            use_tc_tiling_on_sc=False, needs_layout_passes=False),
        out_type=(
            jax.ShapeDtypeStruct((_N, _D), jnp.float32),
            jax.ShapeDtypeStruct((_N, _D), jnp.float32),
        ),
        scratch_types=[
            pltpu.VMEM((3 * _CN,), jnp.float32),
            pltpu.VMEM((3 * _CN,), jnp.float32),
            pltpu.VMEM((_CN,), jnp.float32),
            pltpu.VMEM((_CN,), jnp.float32),
            pltpu.VMEM((_CN,), jnp.float32),
            pltpu.VMEM((_SKEW * _CN,), jnp.float32),
            pltpu.VMEM((_CN, _D), jnp.float32),
            pltpu.VMEM((_CN, _D), jnp.float32),
            pltpu.SemaphoreType.DMA,
            pltpu.SemaphoreType.DMA,
        ],
    )
    def table_pack(pos_hbm, rest_hbm, mu_hbm, lam_hbm, bend_hbm,
                   ta_hbm, tb_hbm,
                   pos_v, rest_v, mu_v, lam_v, bend_v,
                   ska, ta_v, tb_v, sem_in, sem_out):
        wid = lax.axis_index("s") * 2 + lax.axis_index("c")
        iota = lax.iota(jnp.int32, 16)
        pat3 = iota * 3
        pat17 = iota * _SKEW
        sgn = jnp.where((iota >= 3) & (iota < 9), -1.0, 1.0)

        def do_chunk(c):
            base = c * _CN
            pltpu.async_copy(pos_hbm.at[pl.ds(3 * base, 3 * _CN)], pos_v, sem_in)
            pltpu.async_copy(rest_hbm.at[pl.ds(3 * base, 3 * _CN)], rest_v, sem_in)
            pltpu.async_copy(mu_hbm.at[pl.ds(base, _CN)], mu_v, sem_in)
            pltpu.async_copy(lam_hbm.at[pl.ds(base, _CN)], lam_v, sem_in)
            pltpu.async_copy(bend_hbm.at[pl.ds(base, _CN)], bend_v, sem_in)
            pltpu.make_async_copy(pos_hbm.at[pl.ds(0, 3 * _CN)], pos_v, sem_in).wait()
            pltpu.make_async_copy(rest_hbm.at[pl.ds(0, 3 * _CN)], rest_v, sem_in).wait()
            pltpu.make_async_copy(mu_hbm.at[pl.ds(0, _CN)], mu_v, sem_in).wait()
            pltpu.make_async_copy(lam_hbm.at[pl.ds(0, _CN)], lam_v, sem_in).wait()
            pltpu.make_async_copy(bend_hbm.at[pl.ds(0, _CN)], bend_v, sem_in).wait()

            @plsc.parallel_loop(0, _CN // 16, step=1, unroll=2)
            def _(q):
                nb = q * 16
                skoff = q * (16 * _SKEW) + pat17
                for f, src in ((0, mu_v), (1, lam_v), (2, bend_v)):
                    v = src[pl.ds(nb, 16)] * 0.5
                    plsc.store_scatter(ska, [skoff + f], v)
                for c3 in range(3):
                    vp = plsc.load_gather(pos_v, [q * 48 + pat3 + c3])
                    plsc.store_scatter(ska, [skoff + 3 + c3], vp)
                    vr = plsc.load_gather(rest_v, [q * 48 + pat3 + c3]) * _REST_MULT
                    plsc.store_scatter(ska, [skoff + 6 + c3], vr)
                zero = vp * 0.0
                for f in range(9, _D):
                    plsc.store_scatter(ska, [skoff + f], zero)

            @plsc.parallel_loop(0, _CN, step=1, unroll=8)
            def _(j):
                v = plsc.load_gather(ska, [j * _SKEW + iota])
                ta_v[j] = v
                tb_v[j] = v * sgn

            pltpu.async_copy(ta_v, ta_hbm.at[pl.ds(base, _CN)], sem_out)
            pltpu.async_copy(tb_v, tb_hbm.at[pl.ds(base, _CN)], sem_out)
            pltpu.make_async_copy(ta_v, ta_hbm.at[pl.ds(0, _CN)], sem_out).wait()
            pltpu.make_async_copy(tb_v, tb_hbm.at[pl.ds(0, _CN)], sem_out).wait()

        def round_body(t, carry):
            c = wid + t * _NW

            @pl.when(c < _NCH_TP)
            def _():
                do_chunk(c)

            return carry

        lax.fori_loop(0, _TP_ROUNDS, round_body, 0)

    return table_pack


def _make_gather_kernel():
    mesh = plsc.VectorSubcoreMesh(core_axis_name="c", subcore_axis_name="s")

    @functools.partial(
        pl.kernel,
        mesh=mesh,
        compiler_params=pltpu.CompilerParams(
            use_tc_tiling_on_sc=False, needs_layout_passes=False),
        out_type=(
            jax.ShapeDtypeStruct((_E,), jnp.float32),
            jax.ShapeDtypeStruct((_E,), jnp.float32),
            jax.ShapeDtypeStruct((_E,), jnp.float32),
            jax.ShapeDtypeStruct((_E // 128, 4, 128), jnp.float32),
            jax.ShapeDtypeStruct((_E // 128, 4, 128), jnp.float32),
        ),
        scratch_types=[
            pltpu.VMEM((_C,), jnp.int32), pltpu.VMEM((_C,), jnp.int32),
            pltpu.VMEM((_C,), jnp.int32), pltpu.VMEM((_C,), jnp.int32),
            pltpu.VMEM((_C, _D), jnp.float32), pltpu.VMEM((_C, _D), jnp.float32),
            pltpu.VMEM((_C, _D), jnp.float32), pltpu.VMEM((_C, _D), jnp.float32),
            pltpu.VMEM((_SKEW * _C,), jnp.float32),
            pltpu.VMEM((_C,), jnp.float32), pltpu.VMEM((_C,), jnp.float32),
            pltpu.VMEM((_C,), jnp.float32), pltpu.VMEM((_C,), jnp.float32),
            pltpu.VMEM((_C,), jnp.float32), pltpu.VMEM((_C,), jnp.float32),
            pltpu.VMEM((_BPC, 4, 128), jnp.float32), pltpu.VMEM((_BPC, 4, 128), jnp.float32),
            pltpu.VMEM((_BPC, 4, 128), jnp.float32), pltpu.VMEM((_BPC, 4, 128), jnp.float32),
        ] + [pltpu.SemaphoreType.DMA] * 10,
    )
    def edge_gather(ta_hbm, tb_hbm, ei_hbm,
                    mu_hbm, lam_hbm, bend_hbm, rr_hbm, rp_hbm,
                    is0, is1, id0, id1, ra0, ra1, rb0, rb1, skb,
                    mu0, mu1, la0, la1, be0, be1, rp0, rp1, rr0, rr1,
                    sis0, sis1, sid0, sid1, sga0, sga1, sgb0, sgb1,
                    swb0, swb1):
        idx_s = (is0, is1)
        idx_d = (id0, id1)
        ra = (ra0, ra1)
        rb = (rb0, rb1)
        mu_st = (mu0, mu1)
        lam_st = (la0, la1)
        bend_st = (be0, be1)
        rp_st = (rp0, rp1)
        rr_st = (rr0, rr1)
        sis = (sis0, sis1)
        sid = (sid0, sid1)
        sga = (sga0, sga1)
        sgb = (sgb0, sgb1)
        swb = (swb0, swb1)

        wid = lax.axis_index("s") * 2 + lax.axis_index("c")

        iota = lax.iota(jnp.int32, 16)
        skew_pat = iota * _SKEW

        def chunk_of(t):
            return wid + t * _NW

        def issue_idx(t, p):
            base = chunk_of(t) * _C
            pltpu.async_copy(ei_hbm.at[pl.ds(base, _C)], idx_s[p], sis[p])
            pltpu.async_copy(ei_hbm.at[pl.ds(_E + base, _C)], idx_d[p], sid[p])

        def wait_idx(p):
            pltpu.make_async_copy(ei_hbm.at[pl.ds(0, _C)], idx_s[p], sis[p]).wait()
            pltpu.make_async_copy(ei_hbm.at[pl.ds(0, _C)], idx_d[p], sid[p]).wait()

        def issue_gathers(p):
            for s in range(_NSUB):
                sl = pl.ds(s * _CSUB, _CSUB)
                pltpu.async_copy(ta_hbm.at[idx_s[p].at[sl]], ra[p].at[sl], sga[p])
                pltpu.async_copy(tb_hbm.at[idx_d[p].at[sl]], rb[p].at[sl], sgb[p])

        def wait_gathers(p):
            for s in range(_NSUB):
                sl = pl.ds(s * _CSUB, _CSUB)
                pltpu.make_async_copy(ta_hbm.at[idx_s[p].at[sl]], ra[p].at[sl], sga[p]).wait()
                pltpu.make_async_copy(tb_hbm.at[idx_d[p].at[sl]], rb[p].at[sl], sgb[p]).wait()

        def wb_descs(t, p):
            ch = chunk_of(t)
            base = ch * _C
            return (
                pltpu.make_async_copy(mu_st[p], mu_hbm.at[pl.ds(base, _C)], swb[p]),
                pltpu.make_async_copy(lam_st[p], lam_hbm.at[pl.ds(base, _C)], swb[p]),
                pltpu.make_async_copy(bend_st[p], bend_hbm.at[pl.ds(base, _C)], swb[p]),
                pltpu.make_async_copy(rp_st[p], rp_hbm.at[pl.ds(ch * _BPC, _BPC)], swb[p]),
                pltpu.make_async_copy(rr_st[p], rr_hbm.at[pl.ds(ch * _BPC, _BPC)], swb[p]),
            )

        def issue_wb(t, p):
            for d in wb_descs(t, p):
                d.start()

        def wait_wb(p):
            for d in wb_descs(0, p):
                d.wait()

        issue_idx(0, 0)
        issue_idx(1, 1)
        wait_idx(0)
        issue_gathers(0)

        def outer_body(t2, carry):
            for b in (0, 1):
                t = 2 * t2 + b

                @pl.when(chunk_of(t) < _NCH)
                def _():
                    wait_gathers(b)

                @pl.when(chunk_of(t + 2) < _NCH)
                def _():
                    issue_idx(t + 2, b)

                @pl.when((t >= 1) & (chunk_of(t - 1) < _NCH))
                def _():
                    wait_wb(1 - b)

                @pl.when(chunk_of(t + 1) < _NCH)
                def _():
                    wait_idx(1 - b)
                    issue_gathers(1 - b)

                @pl.when(chunk_of(t) < _NCH)
                def _():
                    rap = ra[b]
                    rbp = rb[b]

                    @plsc.parallel_loop(0, _C, step=1, unroll=8)
                    def _(j):
                        v = rap[j] + rbp[j]
                        plsc.store_scatter(skb, [j * _SKEW + iota], v)

                    mup = mu_st[b]
                    lamp = lam_st[b]
                    bendp = bend_st[b]
                    rpp = rp_st[b]
                    rrp = rr_st[b]

                    @plsc.parallel_loop(0, _C // 16, step=1, unroll=2)
                    def _(q):
                        gbase = q * (16 * _SKEW) + skew_pat
                        obase = q * 16
                        mup[pl.ds(obase, 16)] = plsc.load_gather(skb, [gbase])
                        lamp[pl.ds(obase, 16)] = plsc.load_gather(skb, [gbase + 1])
                        bendp[pl.ds(obase, 16)] = plsc.load_gather(skb, [gbase + 2])
                        b3 = q // 8
                        lane = (q % 8) * 16
                        for c in range(3):
                            vpos = plsc.load_gather(skb, [gbase + (3 + c)])
                            rpp[b3, c, pl.ds(lane, 16)] = vpos
                            vrest = plsc.load_gather(skb, [gbase + (6 + c)])
                            rrp[b3, c, pl.ds(lane, 16)] = vrest

                    issue_wb(t, b)
            return carry

        lax.fori_loop(0, _NT // 2, outer_body, 0)

        @pl.when(chunk_of(_NT - 1) < _NCH)
        def _():
            wait_wb(1)

    return edge_gather


_gather_call_cache = []


def kernel(pos, rest_pos, lame_mu_input, lame_lambda_input,
           bending_coeff_input, edge_index):
    if not _gather_call_cache:
        _gather_call_cache.append((_make_pack_kernel(), _make_gather_kernel()))
    _pack_call, _gather_call = _gather_call_cache[0]
    ta, tb = _pack_call(
        pos.reshape(3 * _N),
        rest_pos.reshape(3 * _N),
        lame_mu_input.reshape(_N),
        lame_lambda_input.reshape(_N),
        bending_coeff_input.reshape(_N),
    )
    mu_e, lam_e, bend_e, rr3, rp3 = _gather_call(
        ta, tb, edge_index.reshape(2 * _E))
    rel_rest = rr3.transpose(0, 2, 1).reshape(_E, 4)[:, :3]
    rel_pos = rp3.transpose(0, 2, 1).reshape(_E, 4)[:, :3]
    return (
        mu_e.reshape(_E, 1),
        lam_e.reshape(_E, 1),
        bend_e.reshape(_E, 1),
        rel_rest,
        rel_pos,
    )


# speedup vs baseline: 8.8219x; 1.0011x over previous
import functools

import jax
import jax.numpy as jnp
from jax import lax
from jax.experimental import pallas as pl
from jax.experimental.pallas import tpu as pltpu
from jax.experimental.pallas import tpu_sc as plsc

_N = 100000
_E = 3200000
_REST_MULT = 1.0
_D = 16
_SKEW = 17
_NW = 32
_C = 512
_BPC = _C // 128
_NCH = _E // _C
_NT = (_NCH + _NW - 1) // _NW
_NSUB = 4
_CSUB = _C // _NSUB


_CN = 2000
_NCH_TP = _N // _CN
_TP_ROUNDS = (_NCH_TP + _NW - 1) // _NW


def _make_pack_kernel():
    mesh = plsc.VectorSubcoreMesh(core_axis_name="c", subcore_axis_name="s")

    @functools.partial(
        pl.kernel,
        mesh=mesh,
        compiler_params=pltpu.CompilerParams(
            use_tc_tiling_on_sc=False, needs_layout_passes=False),
        out_type=(
            jax.ShapeDtypeStruct((_N, _D), jnp.float32),
            jax.ShapeDtypeStruct((_N, _D), jnp.float32),
        ),
        scratch_types=[
            pltpu.VMEM((3 * _CN,), jnp.float32),
            pltpu.VMEM((3 * _CN,), jnp.float32),
            pltpu.VMEM((_CN,), jnp.float32),
            pltpu.VMEM((_CN,), jnp.float32),
            pltpu.VMEM((_CN,), jnp.float32),
            pltpu.VMEM((_SKEW * _CN,), jnp.float32),
            pltpu.VMEM((_CN, _D), jnp.float32),
            pltpu.VMEM((_CN, _D), jnp.float32),
            pltpu.SemaphoreType.DMA,
            pltpu.SemaphoreType.DMA,
        ],
    )
    def table_pack(pos_hbm, rest_hbm, mu_hbm, lam_hbm, bend_hbm,
                   ta_hbm, tb_hbm,
                   pos_v, rest_v, mu_v, lam_v, bend_v,
                   ska, ta_v, tb_v, sem_in, sem_out):
        wid = lax.axis_index("s") * 2 + lax.axis_index("c")
        iota = lax.iota(jnp.int32, 16)
        pat3 = iota * 3
        pat17 = iota * _SKEW
        sgn = jnp.where((iota >= 3) & (iota < 9), -1.0, 1.0)

        def do_chunk(c):
            base = c * _CN
            pltpu.async_copy(pos_hbm.at[pl.ds(3 * base, 3 * _CN)], pos_v, sem_in)
            pltpu.async_copy(rest_hbm.at[pl.ds(3 * base, 3 * _CN)], rest_v, sem_in)
            pltpu.async_copy(mu_hbm.at[pl.ds(base, _CN)], mu_v, sem_in)
            pltpu.async_copy(lam_hbm.at[pl.ds(base, _CN)], lam_v, sem_in)
            pltpu.async_copy(bend_hbm.at[pl.ds(base, _CN)], bend_v, sem_in)
            pltpu.make_async_copy(pos_hbm.at[pl.ds(0, 3 * _CN)], pos_v, sem_in).wait()
            pltpu.make_async_copy(rest_hbm.at[pl.ds(0, 3 * _CN)], rest_v, sem_in).wait()
            pltpu.make_async_copy(mu_hbm.at[pl.ds(0, _CN)], mu_v, sem_in).wait()
            pltpu.make_async_copy(lam_hbm.at[pl.ds(0, _CN)], lam_v, sem_in).wait()
            pltpu.make_async_copy(bend_hbm.at[pl.ds(0, _CN)], bend_v, sem_in).wait()

            @plsc.parallel_loop(0, _CN // 16, step=1, unroll=2)
            def _(q):
                nb = q * 16
                skoff = q * (16 * _SKEW) + pat17
                for f, src in ((0, mu_v), (1, lam_v), (2, bend_v)):
                    v = src[pl.ds(nb, 16)] * 0.5
                    plsc.store_scatter(ska, [skoff + f], v)
                for c3 in range(3):
                    vp = plsc.load_gather(pos_v, [q * 48 + pat3 + c3])
                    plsc.store_scatter(ska, [skoff + 3 + c3], vp)
                    vr = plsc.load_gather(rest_v, [q * 48 + pat3 + c3]) * _REST_MULT
                    plsc.store_scatter(ska, [skoff + 6 + c3], vr)
                zero = vp * 0.0
                for f in range(9, _D):
                    plsc.store_scatter(ska, [skoff + f], zero)

            @plsc.parallel_loop(0, _CN, step=1, unroll=8)
            def _(j):
                v = plsc.load_gather(ska, [j * _SKEW + iota])
                ta_v[j] = v
                tb_v[j] = v * sgn

            pltpu.async_copy(ta_v, ta_hbm.at[pl.ds(base, _CN)], sem_out)
            pltpu.async_copy(tb_v, tb_hbm.at[pl.ds(base, _CN)], sem_out)
            pltpu.make_async_copy(ta_v, ta_hbm.at[pl.ds(0, _CN)], sem_out).wait()
            pltpu.make_async_copy(tb_v, tb_hbm.at[pl.ds(0, _CN)], sem_out).wait()

        def round_body(t, carry):
            c = wid + t * _NW

            @pl.when(c < _NCH_TP)
            def _():
                do_chunk(c)

            return carry

        lax.fori_loop(0, _TP_ROUNDS, round_body, 0)

    return table_pack


def _make_gather_kernel():
    mesh = plsc.VectorSubcoreMesh(core_axis_name="c", subcore_axis_name="s")

    @functools.partial(
        pl.kernel,
        mesh=mesh,
        compiler_params=pltpu.CompilerParams(
            use_tc_tiling_on_sc=False, needs_layout_passes=False),
        out_type=(
            jax.ShapeDtypeStruct((_E,), jnp.float32),
            jax.ShapeDtypeStruct((_E,), jnp.float32),
            jax.ShapeDtypeStruct((_E,), jnp.float32),
            jax.ShapeDtypeStruct((_E // 128, 4, 128), jnp.float32),
            jax.ShapeDtypeStruct((_E // 128, 4, 128), jnp.float32),
        ),
        scratch_types=[
            pltpu.VMEM((_C,), jnp.int32), pltpu.VMEM((_C,), jnp.int32),
            pltpu.VMEM((_C,), jnp.int32), pltpu.VMEM((_C,), jnp.int32),
            pltpu.VMEM((_C, _D), jnp.float32), pltpu.VMEM((_C, _D), jnp.float32),
            pltpu.VMEM((_C, _D), jnp.float32), pltpu.VMEM((_C, _D), jnp.float32),
            pltpu.VMEM((_SKEW * _C,), jnp.float32),
            pltpu.VMEM((_C,), jnp.float32), pltpu.VMEM((_C,), jnp.float32),
            pltpu.VMEM((_C,), jnp.float32), pltpu.VMEM((_C,), jnp.float32),
            pltpu.VMEM((_C,), jnp.float32), pltpu.VMEM((_C,), jnp.float32),
            pltpu.VMEM((_BPC, 4, 128), jnp.float32), pltpu.VMEM((_BPC, 4, 128), jnp.float32),
            pltpu.VMEM((_BPC, 4, 128), jnp.float32), pltpu.VMEM((_BPC, 4, 128), jnp.float32),
        ] + [pltpu.SemaphoreType.DMA] * 10,
    )
    def edge_gather(ta_hbm, tb_hbm, ei_hbm,
                    mu_hbm, lam_hbm, bend_hbm, rr_hbm, rp_hbm,
                    is0, is1, id0, id1, ra0, ra1, rb0, rb1, skb,
                    mu0, mu1, la0, la1, be0, be1, rp0, rp1, rr0, rr1,
                    sis0, sis1, sid0, sid1, sga0, sga1, sgb0, sgb1,
                    swb0, swb1):
        idx_s = (is0, is1)
        idx_d = (id0, id1)
        ra = (ra0, ra1)
        rb = (rb0, rb1)
        mu_st = (mu0, mu1)
        lam_st = (la0, la1)
        bend_st = (be0, be1)
        rp_st = (rp0, rp1)
        rr_st = (rr0, rr1)
        sis = (sis0, sis1)
        sid = (sid0, sid1)
        sga = (sga0, sga1)
        sgb = (sgb0, sgb1)
        swb = (swb0, swb1)

        wid = lax.axis_index("s") * 2 + lax.axis_index("c")

        iota = lax.iota(jnp.int32, 16)
        skew_pat = iota * _SKEW

        def chunk_of(t):
            return wid + t * _NW

        def issue_idx(t, p):
            base = chunk_of(t) * _C
            pltpu.async_copy(ei_hbm.at[pl.ds(base, _C)], idx_s[p], sis[p])
            pltpu.async_copy(ei_hbm.at[pl.ds(_E + base, _C)], idx_d[p], sid[p])

        def wait_idx(p):
            pltpu.make_async_copy(ei_hbm.at[pl.ds(0, _C)], idx_s[p], sis[p]).wait()
            pltpu.make_async_copy(ei_hbm.at[pl.ds(0, _C)], idx_d[p], sid[p]).wait()

        def issue_gathers(p):
            for s in range(_NSUB):
                sl = pl.ds(s * _CSUB, _CSUB)
                pltpu.async_copy(ta_hbm.at[idx_s[p].at[sl]], ra[p].at[sl], sga[p])
                pltpu.async_copy(tb_hbm.at[idx_d[p].at[sl]], rb[p].at[sl], sgb[p])

        def wait_gathers(p):
            for s in range(_NSUB):
                sl = pl.ds(s * _CSUB, _CSUB)
                pltpu.make_async_copy(ta_hbm.at[idx_s[p].at[sl]], ra[p].at[sl], sga[p]).wait()
                pltpu.make_async_copy(tb_hbm.at[idx_d[p].at[sl]], rb[p].at[sl], sgb[p]).wait()

        def wb_descs(t, p):
            ch = chunk_of(t)
            base = ch * _C
            return (
                pltpu.make_async_copy(mu_st[p], mu_hbm.at[pl.ds(base, _C)], swb[p]),
                pltpu.make_async_copy(lam_st[p], lam_hbm.at[pl.ds(base, _C)], swb[p]),
                pltpu.make_async_copy(bend_st[p], bend_hbm.at[pl.ds(base, _C)], swb[p]),
                pltpu.make_async_copy(rp_st[p], rp_hbm.at[pl.ds(ch * _BPC, _BPC)], swb[p]),
                pltpu.make_async_copy(rr_st[p], rr_hbm.at[pl.ds(ch * _BPC, _BPC)], swb[p]),
            )

        def issue_wb(t, p):
            for d in wb_descs(t, p):
                d.start()

        def wait_wb(p):
            for d in wb_descs(0, p):
                d.wait()

        issue_idx(0, 0)
        issue_idx(1, 1)
        wait_idx(0)
        issue_gathers(0)

        def outer_body(t2, carry):
            for b in (0, 1):
                t = 2 * t2 + b

                @pl.when(chunk_of(t) < _NCH)
                def _():
                    wait_gathers(b)

                @pl.when(chunk_of(t + 2) < _NCH)
                def _():
                    issue_idx(t + 2, b)

                @pl.when((t >= 1) & (chunk_of(t - 1) < _NCH))
                def _():
                    wait_wb(1 - b)

                @pl.when(chunk_of(t + 1) < _NCH)
                def _():
                    wait_idx(1 - b)
                    issue_gathers(1 - b)

                @pl.when(chunk_of(t) < _NCH)
                def _():
                    rap = ra[b]
                    rbp = rb[b]

                    @plsc.parallel_loop(0, _C, step=1, unroll=8)
                    def _(j):
                        v = rap[j] + rbp[j]
                        plsc.store_scatter(skb, [j * _SKEW + iota], v)

                    mup = mu_st[b]
                    lamp = lam_st[b]
                    bendp = bend_st[b]
                    rpp = rp_st[b]
                    rrp = rr_st[b]

                    @plsc.parallel_loop(0, _C // 16, step=1, unroll=2)
                    def _(q):
                        gbase = q * (16 * _SKEW) + skew_pat
                        obase = q * 16
                        mup[pl.ds(obase, 16)] = plsc.load_gather(skb, [gbase])
                        lamp[pl.ds(obase, 16)] = plsc.load_gather(skb, [gbase + 1])
                        bendp[pl.ds(obase, 16)] = plsc.load_gather(skb, [gbase + 2])
                        b3 = q // 8
                        lane = (q % 8) * 16
                        for c in range(3):
                            vpos = plsc.load_gather(skb, [gbase + (3 + c)])
                            rpp[b3, c, pl.ds(lane, 16)] = vpos
                            vrest = plsc.load_gather(skb, [gbase + (6 + c)])
                            rrp[b3, c, pl.ds(lane, 16)] = vrest

                    issue_wb(t, b)
            return carry

        lax.fori_loop(0, _NT // 2, outer_body, 0)

        @pl.when(chunk_of(_NT - 1) < _NCH)
        def _():
            wait_wb(1)

    return edge_gather


_gather_call_cache = []


def kernel(pos, rest_pos, lame_mu_input, lame_lambda_input,
           bending_coeff_input, edge_index):
    if not _gather_call_cache:
        _gather_call_cache.append((_make_pack_kernel(), _make_gather_kernel()))
    _pack_call, _gather_call = _gather_call_cache[0]
    ta, tb = _pack_call(
        pos.reshape(3 * _N),
        rest_pos.reshape(3 * _N),
        lame_mu_input.reshape(_N),
        lame_lambda_input.reshape(_N),
        bending_coeff_input.reshape(_N),
    )
    mu_e, lam_e, bend_e, rr3, rp3 = _gather_call(
        ta, tb, edge_index.reshape(2 * _E))
    rel_rest = rr3.transpose(0, 2, 1).reshape(_E, 4)[:, :3]
    rel_pos = rp3.transpose(0, 2, 1).reshape(_E, 4)[:, :3]
    return (
        mu_e.reshape(_E, 1),
        lam_e.reshape(_E, 1),
        bend_e.reshape(_E, 1),
        rel_rest,
        rel_pos,
    )
